# Initial kernel scaffold; baseline (speedup 1.0000x reference)
import jax
import jax.numpy as jnp
from jax.experimental import pallas as pl


def kernel(predicted_locs, predicted_scores, predicted_masks, boxes, labels, masks, priors_cxcy):
    raise NotImplementedError("write your pallas kernel here")


# speedup vs baseline: 19.8107x; 19.8107x over previous
import jax
import jax.numpy as jnp
from jax.experimental import pallas as pl
from jax.experimental.pallas import tpu as pltpu

B = 32
P = 8732
NC = 21
NOBJ = 8
H = 150
W = 150
THRESH = 0.5
NEG_POS = 3


def _match_conf_loc_kernel(scores_ref, locs_ref, priors_ref, boxes_ref,
                           labels_ref, out_ref):
    pcx = priors_ref[0:1, :]
    pcy = priors_ref[1:2, :]
    pw = priors_ref[2:3, :]
    ph = priors_ref[3:4, :]
    px1 = pcx - pw * 0.5
    py1 = pcy - ph * 0.5
    px2 = pcx + pw * 0.5
    py2 = pcy + ph * 0.5
    aprior = (px2 - px1) * (py2 - py1)

    boxes = boxes_ref[...]
    labels = labels_ref[...]
    ii = jax.lax.broadcasted_iota(jnp.int32, (B, P), 1)

    ovs = []
    best = None
    besto = jnp.zeros((B, P), jnp.int32)
    for o in range(NOBJ):
        bx1 = boxes[:, 4 * o + 0:4 * o + 1]
        by1 = boxes[:, 4 * o + 1:4 * o + 2]
        bx2 = boxes[:, 4 * o + 2:4 * o + 3]
        by2 = boxes[:, 4 * o + 3:4 * o + 4]
        iw = jnp.maximum(jnp.minimum(bx2, px2) - jnp.maximum(bx1, px1), 0.0)
        ih = jnp.maximum(jnp.minimum(by2, py2) - jnp.maximum(by1, py1), 0.0)
        inter = iw * ih
        abox = (bx2 - bx1) * (by2 - by1)
        ov = inter / (abox + aprior - inter)
        ovs.append(ov)
        if best is None:
            best = ov
        else:
            upd = ov > best
            besto = jnp.where(upd, o, besto)
            best = jnp.where(upd, ov, best)

    for o in range(NOBJ):
        ov = ovs[o]
        m = jnp.max(ov, axis=1, keepdims=True)
        idx = jnp.min(jnp.where(ov == m, ii, P), axis=1, keepdims=True)
        force = ii == idx
        besto = jnp.where(force, o, besto)
        best = jnp.where(force, 1.0, best)

    lab = jnp.zeros((B, P), jnp.int32)
    bx = [jnp.zeros((B, P), jnp.float32) for _ in range(4)]
    for o in range(NOBJ):
        sel_o = besto == o
        lab = jnp.where(sel_o, labels[:, o:o + 1], lab)
        for c in range(4):
            bx[c] = jnp.where(sel_o, boxes[:, 4 * o + c:4 * o + c + 1], bx[c])
    lab = jnp.where(best < THRESH, 0, lab)
    pos = lab != 0
    posf = pos.astype(jnp.float32)
    n_pos_total = jnp.sum(posf)

    cx = (bx[0] + bx[2]) * 0.5
    cy = (bx[1] + bx[3]) * 0.5
    bw = bx[2] - bx[0]
    bh = bx[3] - bx[1]
    g = [
        (cx - pcx) * 10.0 / pw,
        (cy - pcy) * 10.0 / ph,
        jnp.log(bw / pw) * 5.0,
        jnp.log(bh / ph) * 5.0,
    ]
    l1 = jnp.abs(locs_ref[0] - g[0])
    for c in range(1, 4):
        l1 = l1 + jnp.abs(locs_ref[c] - g[c])
    l1_total = jnp.sum(l1 * posf)

    m = scores_ref[0]
    for c in range(1, NC):
        m = jnp.maximum(m, scores_ref[c])
    z = jnp.zeros((B, P), jnp.float32)
    selsc = jnp.zeros((B, P), jnp.float32)
    for c in range(NC):
        x = scores_ref[c]
        z = z + jnp.exp(x - m)
        selsc = selsc + jnp.where(lab == c, x, 0.0)
    conf_all = jnp.log(z) + m - selsc
    conf_pos_total = jnp.sum(conf_all * posf)
    neg = jnp.where(pos, 0.0, conf_all)

    vb = jax.lax.bitcast_convert_type(neg, jnp.int32)
    k = jnp.sum(pos.astype(jnp.int32), axis=1, keepdims=True) * NEG_POS
    prefix = jnp.zeros((B, 1), jnp.int32)
    for b in range(30, -1, -1):
        cand = prefix | (1 << b)
        cnt = jnp.sum((vb >= cand).astype(jnp.int32), axis=1, keepdims=True)
        prefix = jnp.where(cnt >= k, cand, prefix)
    t = jax.lax.bitcast_convert_type(prefix, jnp.float32)
    gt = neg > t
    cgt = jnp.sum(gt.astype(jnp.float32), axis=1, keepdims=True)
    s_img = (jnp.sum(jnp.where(gt, neg, 0.0), axis=1, keepdims=True)
             + (k.astype(jnp.float32) - cgt) * t)
    s_img = jnp.where(k > 0, s_img, 0.0)
    hard_total = jnp.sum(s_img)

    out_ref[0] = n_pos_total
    out_ref[1] = l1_total
    out_ref[2] = conf_pos_total
    out_ref[3] = hard_total


def _mask_loss_kernel(pm_ref, mk_ref, out_ref):
    i = pl.program_id(0)

    @pl.when(i == 0)
    def _init():
        out_ref[0] = 0.0

    mk = mk_ref[0]
    m = pm_ref[0, 0]
    for c in range(1, NC):
        m = jnp.maximum(m, pm_ref[0, c])
    z = jnp.zeros((H, W), jnp.float32)
    sel = jnp.zeros((H, W), jnp.float32)
    for c in range(NC):
        x = pm_ref[0, c]
        z = z + jnp.exp(x - m)
        sel = sel + jnp.where(mk == c, x, 0.0)
    logz = jnp.log(z) + m
    out_ref[0] += jnp.sum(logz - sel)


def kernel(predicted_locs, predicted_scores, predicted_masks, boxes, labels,
           masks, priors_cxcy):
    scores_t = jnp.moveaxis(predicted_scores, 2, 0)
    locs_t = jnp.moveaxis(predicted_locs, 2, 0)
    priors_t = priors_cxcy.T
    boxes2 = boxes.reshape(B, NOBJ * 4)
    labels_i = labels.astype(jnp.int32)
    masks_i = masks.astype(jnp.int32)

    stats = pl.pallas_call(
        _match_conf_loc_kernel,
        out_shape=jax.ShapeDtypeStruct((4,), jnp.float32),
        in_specs=[
            pl.BlockSpec(memory_space=pltpu.VMEM),
            pl.BlockSpec(memory_space=pltpu.VMEM),
            pl.BlockSpec(memory_space=pltpu.VMEM),
            pl.BlockSpec(memory_space=pltpu.VMEM),
            pl.BlockSpec(memory_space=pltpu.VMEM),
        ],
        out_specs=pl.BlockSpec(memory_space=pltpu.SMEM),
    )(scores_t, locs_t, priors_t, boxes2, labels_i)

    msum = pl.pallas_call(
        _mask_loss_kernel,
        grid=(B,),
        out_shape=jax.ShapeDtypeStruct((1,), jnp.float32),
        in_specs=[
            pl.BlockSpec((1, NC, H, W), lambda i: (i, 0, 0, 0)),
            pl.BlockSpec((1, H, W), lambda i: (i, 0, 0)),
        ],
        out_specs=pl.BlockSpec(memory_space=pltpu.SMEM),
    )(predicted_masks, masks_i)

    n_pos_total = stats[0]
    loc_loss = stats[1] / (n_pos_total * 4.0)
    conf_loss = (stats[3] + stats[2]) / n_pos_total
    mask_loss = msum[0] / float(H * W) / float(B)
    return conf_loss + loc_loss + mask_loss
